# Initial kernel scaffold; baseline (speedup 1.0000x reference)
import functools

import jax
import jax.numpy as jnp
from jax import lax
from jax.experimental import pallas as pl
from jax.experimental.pallas import tpu as pltpu
from jax.experimental.pallas import tpu_sc as plsc

N_USERS = 60000
N_ITEMS = 40000
N = N_USERS + N_ITEMS
NP = 100096
PAD_NODE = NP - 1
E = 1600000
EP = 1601536
EROWS = EP // 128
G = 17
TPR = EROWS // 16
NGE = TPR // G
DROWS = EROWS // 2
DTPR = DROWS // 16
NGD = DTPR // G
STRIPE = NP // 16
BLK = 256
NBLK = NP // BLK

_mesh = plsc.VectorSubcoreMesh(core_axis_name="c", subcore_axis_name="s")

_f32 = jnp.float32
_i32 = jnp.int32


def _zero_vec_ref(ref, n):
  zv = jnp.zeros((16,), _f32)

  def body(i, _):
    ref[pl.ds(i * 16, 16)] = zv
    return 0

  lax.fori_loop(0, n // 16, body, 0)


@functools.partial(
    pl.kernel,
    out_type=(
        jax.ShapeDtypeStruct((NP,), _f32),
        jax.ShapeDtypeStruct((NP,), _f32),
    ),
    mesh=_mesh,
    scratch_types=[
        pltpu.VMEM_SHARED((NP,), _f32),
        pltpu.VMEM((G, 128), _i32),
        pltpu.VMEM((128,), _f32),
        pltpu.VMEM((512,), _f32),
        pltpu.SemaphoreType.DMA,
    ],
)
def _deg_kernel(dst2, d0_out, d1_out, deg_sp, didx, ones_v, zbuf, sem):
  c = lax.axis_index("c")
  s = lax.axis_index("s")
  r0 = s * STRIPE

  _zero_vec_ref(zbuf, 512)

  def fill_ones(i, _):
    ones_v[pl.ds(i * 16, 16)] = jnp.ones((16,), _f32)
    return 0

  lax.fori_loop(0, 8, fill_ones, 0)

  for j in range(12):
    pltpu.make_async_copy(zbuf, deg_sp.at[pl.ds(r0 + j * 512, 512)], sem).start()
  pltpu.make_async_copy(
      zbuf.at[pl.ds(0, 112)], deg_sp.at[pl.ds(r0 + 12 * 512, 112)], sem
  ).start()
  for j in range(12):
    pltpu.make_async_copy(zbuf, deg_sp.at[pl.ds(r0 + j * 512, 512)], sem).wait()
  pltpu.make_async_copy(
      zbuf.at[pl.ds(0, 112)], deg_sp.at[pl.ds(r0 + 12 * 512, 112)], sem
  ).wait()

  plsc.subcore_barrier()

  base = c * DROWS + s * DTPR

  def do_group(g, drain):
    if drain:
      for j in range(G):
        pltpu.make_async_copy(ones_v, deg_sp.at[didx.at[j]], sem).wait()
    pltpu.sync_copy(dst2.at[pl.ds(base + g * G, G)], didx)
    for j in range(G):
      pltpu.make_async_copy(ones_v, deg_sp.at[didx.at[j]], sem).start(add=True)

  do_group(0, False)

  def loop_body(g, _):
    do_group(g, True)
    return 0

  lax.fori_loop(1, NGD, loop_body, 0)

  for j in range(G):
    pltpu.make_async_copy(ones_v, deg_sp.at[didx.at[j]], sem).wait()

  plsc.subcore_barrier()

  def writeout(dout):
    for j in range(12):
      pltpu.make_async_copy(
          deg_sp.at[pl.ds(r0 + j * 512, 512)], dout.at[pl.ds(r0 + j * 512, 512)],
          sem).start()
    pltpu.make_async_copy(
        deg_sp.at[pl.ds(r0 + 12 * 512, 112)],
        dout.at[pl.ds(r0 + 12 * 512, 112)], sem).start()
    for j in range(12):
      pltpu.make_async_copy(
          deg_sp.at[pl.ds(r0 + j * 512, 512)], dout.at[pl.ds(r0 + j * 512, 512)],
          sem).wait()
    pltpu.make_async_copy(
        deg_sp.at[pl.ds(r0 + 12 * 512, 112)],
        dout.at[pl.ds(r0 + 12 * 512, 112)], sem).wait()

  @pl.when(c == 0)
  def _():
    writeout(d0_out)

  @pl.when(c == 1)
  def _():
    writeout(d1_out)


@functools.partial(
    pl.kernel,
    out_type=jax.ShapeDtypeStruct((2, NP, 16), _f32),
    mesh=_mesh,
    scratch_types=[
        pltpu.VMEM_SHARED((NP, 16), _f32),
        pltpu.VMEM((2, G, 128), _i32),
        pltpu.VMEM((2, G, 128), _i32),
        pltpu.VMEM((2, G, 128, 16), _f32),
        pltpu.VMEM((128, 16), _f32),
        pltpu.SemaphoreType.DMA,
        pltpu.SemaphoreType.DMA,
        pltpu.SemaphoreType.DMA,
    ],
)
def _edge_kernel(src2, dst2, z0, z1, w_out, w_sp, sidx, didx, rbuf, zrow,
                 sem_g, sem_s0, sem_s1):
  c = lax.axis_index("c")
  s = lax.axis_index("s")
  r0 = s * STRIPE

  zv = jnp.zeros((16,), _f32)

  def zbody(i, _):
    zrow[i] = zv
    return 0

  lax.fori_loop(0, 128, zbody, 0)

  for j in range(48):
    pltpu.make_async_copy(zrow, w_sp.at[pl.ds(r0 + j * 128, 128)], sem_g).start()
  pltpu.make_async_copy(
      zrow.at[pl.ds(0, 112)], w_sp.at[pl.ds(r0 + 48 * 128, 112)], sem_g).start()
  for j in range(48):
    pltpu.make_async_copy(zrow, w_sp.at[pl.ds(r0 + j * 128, 128)], sem_g).wait()
  pltpu.make_async_copy(
      zrow.at[pl.ds(0, 112)], w_sp.at[pl.ds(r0 + 48 * 128, 112)], sem_g).wait()

  plsc.subcore_barrier()

  base = s * TPR
  sems = (sem_s0, sem_s1)

  def do_group(g, b, drain):
    sem_s = sems[b]
    if drain:
      for j in range(G):
        pltpu.make_async_copy(rbuf.at[b, j], w_sp.at[didx.at[b, j]], sem_s).wait()
    pltpu.sync_copy(src2.at[pl.ds(base + g * G, G)], sidx.at[b])
    pltpu.sync_copy(dst2.at[pl.ds(base + g * G, G)], didx.at[b])

    @pl.when(c == 0)
    def _():
      for j in range(G):
        pltpu.make_async_copy(z0.at[sidx.at[b, j]], rbuf.at[b, j], sem_g).start()

    @pl.when(c == 1)
    def _():
      for j in range(G):
        pltpu.make_async_copy(z1.at[sidx.at[b, j]], rbuf.at[b, j], sem_g).start()

    for j in range(G):
      pltpu.make_async_copy(z0.at[sidx.at[b, j]], rbuf.at[b, j], sem_g).wait()
    for j in range(G):
      pltpu.make_async_copy(
          rbuf.at[b, j], w_sp.at[didx.at[b, j]], sem_s).start(add=True)

  do_group(0, 0, False)
  do_group(1, 1, False)

  def loop_body(p, _):
    do_group(2 * p, 0, True)
    do_group(2 * p + 1, 1, True)
    return 0

  lax.fori_loop(1, NGE // 2, loop_body, 0)

  for b in range(2):
    for j in range(G):
      pltpu.make_async_copy(rbuf.at[b, j], w_sp.at[didx.at[b, j]], sems[b]).wait()

  plsc.subcore_barrier()

  for j in range(48):
    pltpu.make_async_copy(
        w_sp.at[pl.ds(r0 + j * 128, 128)],
        w_out.at[c, pl.ds(r0 + j * 128, 128)], sem_g).start()
  pltpu.make_async_copy(
      w_sp.at[pl.ds(r0 + 48 * 128, 112)],
      w_out.at[c, pl.ds(r0 + 48 * 128, 112)], sem_g).start()
  for j in range(48):
    pltpu.make_async_copy(
        w_sp.at[pl.ds(r0 + j * 128, 128)],
        w_out.at[c, pl.ds(r0 + j * 128, 128)], sem_g).wait()
  pltpu.make_async_copy(
      w_sp.at[pl.ds(r0 + 48 * 128, 112)],
      w_out.at[c, pl.ds(r0 + 48 * 128, 112)], sem_g).wait()


def _tc_init_body(d0, d1, x, dis_o, z0_o, z1_o, avg_o):
  deg = d0[...] + d1[...]
  dis = jnp.where(deg > 0.0, lax.rsqrt(jnp.maximum(deg, 1e-12)), 0.0)
  dis_o[...] = dis
  d2 = jnp.reshape(dis, (BLK, 1))
  xv = x[...]
  z0_o[...] = xv[:, :16] * d2
  z1_o[...] = xv[:, 16:] * d2
  avg_o[...] = xv * 0.25


def _tc_init(d0, d1, x0):
  return pl.pallas_call(
      _tc_init_body,
      grid=(NBLK,),
      in_specs=[
          pl.BlockSpec((BLK,), lambda i: (i,)),
          pl.BlockSpec((BLK,), lambda i: (i,)),
          pl.BlockSpec((BLK, 32), lambda i: (i, 0)),
      ],
      out_specs=[
          pl.BlockSpec((BLK,), lambda i: (i,)),
          pl.BlockSpec((BLK, 16), lambda i: (i, 0)),
          pl.BlockSpec((BLK, 16), lambda i: (i, 0)),
          pl.BlockSpec((BLK, 32), lambda i: (i, 0)),
      ],
      out_shape=[
          jax.ShapeDtypeStruct((NP,), _f32),
          jax.ShapeDtypeStruct((NP, 16), _f32),
          jax.ShapeDtypeStruct((NP, 16), _f32),
          jax.ShapeDtypeStruct((NP, 32), _f32),
      ],
  )(d0, d1, x0)


def _tc_scale_body(w, dis, avg_in, z0_o, z1_o, avg_o):
  wv = w[...]
  d = jnp.reshape(dis[...], (BLK, 1))
  d2 = d * d
  w0 = wv[0]
  w1 = wv[1]
  z0_o[...] = w0 * d2
  z1_o[...] = w1 * d2
  wcat = jnp.concatenate([w0, w1], axis=1)
  avg_o[...] = avg_in[...] + wcat * (d * 0.25)


def _tc_scale(w3, dis, avg):
  return pl.pallas_call(
      _tc_scale_body,
      grid=(NBLK,),
      in_specs=[
          pl.BlockSpec((2, BLK, 16), lambda i: (0, i, 0)),
          pl.BlockSpec((BLK,), lambda i: (i,)),
          pl.BlockSpec((BLK, 32), lambda i: (i, 0)),
      ],
      out_specs=[
          pl.BlockSpec((BLK, 16), lambda i: (i, 0)),
          pl.BlockSpec((BLK, 16), lambda i: (i, 0)),
          pl.BlockSpec((BLK, 32), lambda i: (i, 0)),
      ],
      out_shape=[
          jax.ShapeDtypeStruct((NP, 16), _f32),
          jax.ShapeDtypeStruct((NP, 16), _f32),
          jax.ShapeDtypeStruct((NP, 32), _f32),
      ],
  )(w3, dis, avg)


def kernel(user_emb, item_emb, edge_index):
  x0 = jnp.concatenate([user_emb, item_emb], axis=0)
  x0 = jnp.pad(x0, ((0, NP - N), (0, 0)))
  pad = jnp.full((EP - E,), PAD_NODE, dtype=_i32)
  src2 = jnp.concatenate([edge_index[0], pad]).reshape(EROWS, 128)
  dst2 = jnp.concatenate([edge_index[1], pad]).reshape(EROWS, 128)

  d0, d1 = _deg_kernel(dst2)
  dis, z0, z1, avg = _tc_init(d0, d1, x0)
  for _ in range(3):
    w3 = _edge_kernel(src2, dst2, z0, z1)
    z0, z1, avg = _tc_scale(w3, dis, avg)

  return avg[:N_USERS, :], avg[N_USERS:N, :]


# speedup vs baseline: 18.1769x; 18.1769x over previous
import functools

import jax
import jax.numpy as jnp
from jax import lax
from jax.experimental import pallas as pl
from jax.experimental.pallas import tpu as pltpu
from jax.experimental.pallas import tpu_sc as plsc

N_USERS = 60000
N_ITEMS = 40000
N = N_USERS + N_ITEMS
NP = 102400
PAD_NODE = NP - 1
E = 1600000
EP = 1638400
EROWS = EP // 128
TPR = EROWS // 16
SUP = 8
NSG = TPR // SUP
G = 4
DG = 16
DROWS = EROWS // 2
DTPR = DROWS // 16
NGD = DTPR // DG
STRIPE = NP // 16
BLK = 2048
NBLK = NP // BLK

_mesh = plsc.VectorSubcoreMesh(core_axis_name="c", subcore_axis_name="s")

_f32 = jnp.float32
_i32 = jnp.int32


def _zero_vec_ref(ref, n):
  zv = jnp.zeros((16,), _f32)

  def body(i, _):
    ref[pl.ds(i * 16, 16)] = zv
    return 0

  lax.fori_loop(0, n // 16, body, 0)


@functools.partial(
    pl.kernel,
    out_type=(
        jax.ShapeDtypeStruct((NP,), _f32),
        jax.ShapeDtypeStruct((NP,), _f32),
    ),
    mesh=_mesh,
    compiler_params=pltpu.CompilerParams(use_tc_tiling_on_sc=False),
    scratch_types=[
        pltpu.VMEM_SHARED((NP,), _f32),
        pltpu.VMEM((DG, 128), _i32),
        pltpu.VMEM((128,), _f32),
        pltpu.VMEM((512,), _f32),
        pltpu.SemaphoreType.DMA,
    ],
)
def _deg_kernel(dst2, d0_out, d1_out, deg_sp, didx, ones_v, zbuf, sem):
  c = lax.axis_index("c")
  s = lax.axis_index("s")
  r0 = s * STRIPE

  _zero_vec_ref(zbuf, 512)

  def fill_ones(i, _):
    ones_v[pl.ds(i * 16, 16)] = jnp.ones((16,), _f32)
    return 0

  lax.fori_loop(0, 8, fill_ones, 0)

  for j in range(12):
    pltpu.make_async_copy(zbuf, deg_sp.at[pl.ds(r0 + j * 512, 512)], sem).start()
  pltpu.make_async_copy(
      zbuf.at[pl.ds(0, 256)], deg_sp.at[pl.ds(r0 + 12 * 512, 256)], sem
  ).start()
  for j in range(12):
    pltpu.make_async_copy(zbuf, deg_sp.at[pl.ds(r0 + j * 512, 512)], sem).wait()
  pltpu.make_async_copy(
      zbuf.at[pl.ds(0, 256)], deg_sp.at[pl.ds(r0 + 12 * 512, 256)], sem
  ).wait()

  plsc.subcore_barrier()

  base = c * DROWS + s * DTPR

  def do_group(g, drain):
    if drain:
      for j in range(DG):
        pltpu.make_async_copy(ones_v, deg_sp.at[didx.at[j]], sem).wait()
    pltpu.sync_copy(dst2.at[pl.ds(base + g * DG, DG)], didx)
    for j in range(DG):
      pltpu.make_async_copy(ones_v, deg_sp.at[didx.at[j]], sem).start(add=True)

  do_group(0, False)

  def loop_body(g, _):
    do_group(g, True)
    return 0

  lax.fori_loop(1, NGD, loop_body, 0)

  for j in range(DG):
    pltpu.make_async_copy(ones_v, deg_sp.at[didx.at[j]], sem).wait()

  plsc.subcore_barrier()

  def writeout(dout):
    for j in range(12):
      pltpu.sync_copy(deg_sp.at[pl.ds(r0 + j * 512, 512)], zbuf)
      pltpu.sync_copy(zbuf, dout.at[pl.ds(r0 + j * 512, 512)])
    pltpu.sync_copy(deg_sp.at[pl.ds(r0 + 12 * 512, 256)], zbuf.at[pl.ds(0, 256)])
    pltpu.sync_copy(zbuf.at[pl.ds(0, 256)], dout.at[pl.ds(r0 + 12 * 512, 256)])

  @pl.when(c == 0)
  def _():
    writeout(d0_out)

  @pl.when(c == 1)
  def _():
    writeout(d1_out)


@functools.partial(
    pl.kernel,
    out_type=jax.ShapeDtypeStruct((2, NP, 16), _f32),
    mesh=_mesh,
    compiler_params=pltpu.CompilerParams(use_tc_tiling_on_sc=False),
    scratch_types=[
        pltpu.VMEM_SHARED((NP, 16), _f32),
        pltpu.VMEM((3, 2 * SUP, 128), _i32),
        pltpu.VMEM((2, 512, 16), _f32),
        pltpu.SemaphoreType.DMA,
        pltpu.SemaphoreType.DMA,
        pltpu.SemaphoreType.DMA,
        pltpu.SemaphoreType.DMA,
        pltpu.SemaphoreType.DMA,
        pltpu.SemaphoreType.DMA,
    ],
)
def _edge_kernel(sd2, z0, z1, w_out, w_sp, ibuf, rbuf,
                 sem_i0, sem_i1, sem_i2, sem_g, sem_s0, sem_s1):
  c = lax.axis_index("c")
  s = lax.axis_index("s")
  r0 = s * STRIPE

  zv = jnp.zeros((16,), _f32)

  def zbody(i, _):
    rbuf[0, i] = zv
    return 0

  lax.fori_loop(0, 128, zbody, 0)

  zsrc = rbuf.at[0, pl.ds(0, 128)]
  for j in range(50):
    pltpu.make_async_copy(zsrc, w_sp.at[pl.ds(r0 + j * 128, 128)], sem_g).start()
  for j in range(50):
    pltpu.make_async_copy(zsrc, w_sp.at[pl.ds(r0 + j * 128, 128)], sem_g).wait()

  plsc.subcore_barrier()

  base = s * TPR
  sems = (sem_s0, sem_s1)
  isems = (sem_i0, sem_i1, sem_i2)

  def idx_fetch(sg, slot):
    row = 2 * base + 2 * SUP * jnp.where(sg < NSG, sg, 0)
    pltpu.make_async_copy(
        sd2.at[pl.ds(row, 2 * SUP)], ibuf.at[slot], isems[slot]).start()

  def do_sg(sg, slot, drain):
    pltpu.make_async_copy(
        sd2.at[pl.ds(0, 2 * SUP)], ibuf.at[slot], isems[slot]).wait()
    prev = (slot + 2) % 3
    if drain:
      for h in range(2):
        for j in range(G):
          pltpu.make_async_copy(
              rbuf.at[h, pl.ds(j * 128, 128)],
              w_sp.at[ibuf.at[prev, 2 * (h * G + j) + 1]], sems[h]).wait()

    @pl.when(c == 0)
    def _():
      for j in range(2 * G):
        pltpu.make_async_copy(
            z0.at[ibuf.at[slot, 2 * j]],
            rbuf.at[j // G, pl.ds((j % G) * 128, 128)], sem_g).start()

    @pl.when(c == 1)
    def _():
      for j in range(2 * G):
        pltpu.make_async_copy(
            z1.at[ibuf.at[slot, 2 * j]],
            rbuf.at[j // G, pl.ds((j % G) * 128, 128)], sem_g).start()

    idx_fetch(sg + 2, prev)

    for j in range(2 * G):
      pltpu.make_async_copy(
          z0.at[ibuf.at[slot, 2 * j]],
          rbuf.at[j // G, pl.ds((j % G) * 128, 128)], sem_g).wait()
    for h in range(2):
      for j in range(G):
        pltpu.make_async_copy(
            rbuf.at[h, pl.ds(j * 128, 128)],
            w_sp.at[ibuf.at[slot, 2 * (h * G + j) + 1]], sems[h]).start(add=True)

  idx_fetch(0, 0)
  idx_fetch(1, 1)
  do_sg(0, 0, False)

  def loop_body(q, _):
    do_sg(3 * q + 1, 1, True)
    do_sg(3 * q + 2, 2, True)
    do_sg(3 * q + 3, 0, True)
    return 0

  lax.fori_loop(0, (NSG - 1) // 3, loop_body, 0)

  for h in range(2):
    for j in range(G):
      pltpu.make_async_copy(
          rbuf.at[h, pl.ds(j * 128, 128)],
          w_sp.at[ibuf.at[0, 2 * (h * G + j) + 1]], sems[h]).wait()
  pltpu.make_async_copy(sd2.at[pl.ds(0, 2 * SUP)], ibuf.at[1], sem_i1).wait()
  pltpu.make_async_copy(sd2.at[pl.ds(0, 2 * SUP)], ibuf.at[2], sem_i2).wait()

  plsc.subcore_barrier()

  for j in range(12):
    pltpu.sync_copy(w_sp.at[pl.ds(r0 + j * 512, 512)], rbuf.at[0])
    pltpu.sync_copy(rbuf.at[0], w_out.at[c, pl.ds(r0 + j * 512, 512)])
  pltpu.sync_copy(
      w_sp.at[pl.ds(r0 + 12 * 512, 256)], rbuf.at[0, pl.ds(0, 256)])
  pltpu.sync_copy(
      rbuf.at[0, pl.ds(0, 256)], w_out.at[c, pl.ds(r0 + 12 * 512, 256)])


def _tc_init_body(d0, d1, x, dis_o, z0_o, z1_o, avg_o):
  deg = d0[...] + d1[...]
  dis = jnp.where(deg > 0.0, lax.rsqrt(jnp.maximum(deg, 1e-12)), 0.0)
  dis_o[...] = dis
  d2 = jnp.reshape(dis, (BLK, 1))
  xv = x[...]
  z0_o[...] = xv[:, :16] * d2
  z1_o[...] = xv[:, 16:] * d2
  avg_o[...] = xv * 0.25


def _tc_init(d0, d1, x0):
  return pl.pallas_call(
      _tc_init_body,
      grid=(NBLK,),
      in_specs=[
          pl.BlockSpec((BLK,), lambda i: (i,)),
          pl.BlockSpec((BLK,), lambda i: (i,)),
          pl.BlockSpec((BLK, 32), lambda i: (i, 0)),
      ],
      out_specs=[
          pl.BlockSpec((BLK,), lambda i: (i,)),
          pl.BlockSpec((BLK, 16), lambda i: (i, 0)),
          pl.BlockSpec((BLK, 16), lambda i: (i, 0)),
          pl.BlockSpec((BLK, 32), lambda i: (i, 0)),
      ],
      out_shape=[
          jax.ShapeDtypeStruct((NP,), _f32),
          jax.ShapeDtypeStruct((NP, 16), _f32),
          jax.ShapeDtypeStruct((NP, 16), _f32),
          jax.ShapeDtypeStruct((NP, 32), _f32),
      ],
  )(d0, d1, x0)


def _tc_scale_body(w, dis, avg_in, z0_o, z1_o, avg_o):
  wv = w[...]
  d = jnp.reshape(dis[...], (BLK, 1))
  d2 = d * d
  w0 = wv[0]
  w1 = wv[1]
  z0_o[...] = w0 * d2
  z1_o[...] = w1 * d2
  wcat = jnp.concatenate([w0, w1], axis=1)
  avg_o[...] = avg_in[...] + wcat * (d * 0.25)


def _tc_scale(w3, dis, avg):
  return pl.pallas_call(
      _tc_scale_body,
      grid=(NBLK,),
      in_specs=[
          pl.BlockSpec((2, BLK, 16), lambda i: (0, i, 0)),
          pl.BlockSpec((BLK,), lambda i: (i,)),
          pl.BlockSpec((BLK, 32), lambda i: (i, 0)),
      ],
      out_specs=[
          pl.BlockSpec((BLK, 16), lambda i: (i, 0)),
          pl.BlockSpec((BLK, 16), lambda i: (i, 0)),
          pl.BlockSpec((BLK, 32), lambda i: (i, 0)),
      ],
      out_shape=[
          jax.ShapeDtypeStruct((NP, 16), _f32),
          jax.ShapeDtypeStruct((NP, 16), _f32),
          jax.ShapeDtypeStruct((NP, 32), _f32),
      ],
  )(w3, dis, avg)


def kernel(user_emb, item_emb, edge_index):
  x0 = jnp.concatenate([user_emb, item_emb], axis=0)
  x0 = jnp.pad(x0, ((0, NP - N), (0, 0)))
  pad = jnp.full((EP - E,), PAD_NODE, dtype=_i32)
  src2 = jnp.concatenate([edge_index[0], pad]).reshape(EROWS, 128)
  dst2 = jnp.concatenate([edge_index[1], pad]).reshape(EROWS, 128)
  sd2 = jnp.stack([src2, dst2], axis=1).reshape(2 * EROWS, 128)

  d0, d1 = _deg_kernel(dst2)
  dis, z0, z1, avg = _tc_init(d0, d1, x0)
  for _ in range(3):
    w3 = _edge_kernel(sd2, z0, z1)
    z0, z1, avg = _tc_scale(w3, dis, avg)

  return avg[:N_USERS, :], avg[N_USERS:N, :]


# speedup vs baseline: 20.5573x; 1.1310x over previous
import functools

import jax
import jax.numpy as jnp
from jax import lax
from jax.experimental import pallas as pl
from jax.experimental.pallas import tpu as pltpu
from jax.experimental.pallas import tpu_sc as plsc

N_USERS = 60000
N_ITEMS = 40000
N = N_USERS + N_ITEMS
NP = 100096
PAD_NODE = NP - 1
E = 1600000
EP = 1638400
EROWS = EP // 128
TPR = EROWS // 16
SUP = 8
NSG = TPR // SUP
G = 4
DG = 16
DROWS = EROWS // 2
DTPR = DROWS // 16
NGD = DTPR // DG
STRIPE = NP // 16
NCH = 12

_mesh = plsc.VectorSubcoreMesh(core_axis_name="c", subcore_axis_name="s")

_f32 = jnp.float32
_i32 = jnp.int32


def _zero_vec_ref(ref, n):
  zv = jnp.zeros((16,), _f32)

  def body(i, _):
    ref[pl.ds(i * 16, 16)] = zv
    return 0

  lax.fori_loop(0, n // 16, body, 0)


def _rsqrt16(x):
  xh = x * 0.5
  i = plsc.bitcast(x, _i32)
  i = 0x5F3759DF - (i >> 1)
  y = plsc.bitcast(i, _f32)
  y = y * (1.5 - xh * y * y)
  y = y * (1.5 - xh * y * y)
  y = y * (1.5 - xh * y * y)
  return jnp.where(x > 0.0, y, 0.0)


def _splat(ref, i):
  return plsc.load_gather(ref, [jnp.full((16,), i, _i32)])


@functools.partial(
    pl.kernel,
    out_type=(
        jax.ShapeDtypeStruct((NP,), _f32),
        jax.ShapeDtypeStruct((NP,), _f32),
    ),
    mesh=_mesh,
    compiler_params=pltpu.CompilerParams(use_tc_tiling_on_sc=False),
    scratch_types=[
        pltpu.VMEM_SHARED((NP,), _f32),
        pltpu.VMEM((DG, 128), _i32),
        pltpu.VMEM((128,), _f32),
        pltpu.VMEM((512,), _f32),
        pltpu.SemaphoreType.DMA,
    ],
)
def _deg_kernel(dst2, d0_out, d1_out, deg_sp, didx, ones_v, zbuf, sem):
  c = lax.axis_index("c")
  s = lax.axis_index("s")
  r0 = s * STRIPE

  _zero_vec_ref(zbuf, 512)

  def fill_ones(i, _):
    ones_v[pl.ds(i * 16, 16)] = jnp.ones((16,), _f32)
    return 0

  lax.fori_loop(0, 8, fill_ones, 0)

  for j in range(NCH):
    pltpu.make_async_copy(zbuf, deg_sp.at[pl.ds(r0 + j * 512, 512)], sem).start()
  pltpu.make_async_copy(
      zbuf.at[pl.ds(0, 112)], deg_sp.at[pl.ds(r0 + NCH * 512, 112)], sem
  ).start()
  for j in range(NCH):
    pltpu.make_async_copy(zbuf, deg_sp.at[pl.ds(r0 + j * 512, 512)], sem).wait()
  pltpu.make_async_copy(
      zbuf.at[pl.ds(0, 112)], deg_sp.at[pl.ds(r0 + NCH * 512, 112)], sem
  ).wait()

  plsc.subcore_barrier()

  base = c * DROWS + s * DTPR

  def do_group(g, drain):
    if drain:
      for j in range(DG):
        pltpu.make_async_copy(ones_v, deg_sp.at[didx.at[j]], sem).wait()
    pltpu.sync_copy(dst2.at[pl.ds(base + g * DG, DG)], didx)
    for j in range(DG):
      pltpu.make_async_copy(ones_v, deg_sp.at[didx.at[j]], sem).start(add=True)

  do_group(0, False)

  def loop_body(g, _):
    do_group(g, True)
    return 0

  lax.fori_loop(1, NGD, loop_body, 0)

  for j in range(DG):
    pltpu.make_async_copy(ones_v, deg_sp.at[didx.at[j]], sem).wait()

  plsc.subcore_barrier()

  def writeout(dout):
    for j in range(NCH):
      pltpu.sync_copy(deg_sp.at[pl.ds(r0 + j * 512, 512)], zbuf)
      pltpu.sync_copy(zbuf, dout.at[pl.ds(r0 + j * 512, 512)])
    pltpu.sync_copy(deg_sp.at[pl.ds(r0 + NCH * 512, 112)], zbuf.at[pl.ds(0, 112)])
    pltpu.sync_copy(zbuf.at[pl.ds(0, 112)], dout.at[pl.ds(r0 + NCH * 512, 112)])

  @pl.when(c == 0)
  def _():
    writeout(d0_out)

  @pl.when(c == 1)
  def _():
    writeout(d1_out)


CH = 368
NCHV = STRIPE // CH


@functools.partial(
    pl.kernel,
    out_type=(
        jax.ShapeDtypeStruct((2, NP, 16), _f32),
        jax.ShapeDtypeStruct((4, 2, NP, 16), _f32),
    ),
    mesh=_mesh,
    compiler_params=pltpu.CompilerParams(
        use_tc_tiling_on_sc=False, needs_layout_passes=False),
    scratch_types=[
        pltpu.VMEM_SHARED((NP, 16), _f32),
        pltpu.VMEM((3, 2 * SUP, 128), _i32),
        pltpu.VMEM((2, 512, 16), _f32),
        pltpu.VMEM((128, 16), _f32),
        pltpu.VMEM((512,), _f32),
        pltpu.VMEM((512,), _f32),
        pltpu.VMEM((512,), _f32),
        pltpu.SemaphoreType.DMA,
        pltpu.SemaphoreType.DMA,
        pltpu.SemaphoreType.DMA,
        pltpu.SemaphoreType.DMA,
        pltpu.SemaphoreType.DMA,
        pltpu.SemaphoreType.DMA,
    ],
)
def _mega_kernel(sd2, d0, d1, x2, avg, zall,
                 w_sp, ibuf, rbuf, zrow, t0, t1, dbuf,
                 sem_i0, sem_i1, sem_i2, sem_g, sem_s0, sem_s1):
  c = lax.axis_index("c")
  s = lax.axis_index("s")
  r0 = s * STRIPE

  zv = jnp.zeros((16,), _f32)

  def zbody(i, _):
    zrow[i] = zv
    return 0

  lax.fori_loop(0, 128, zbody, 0)

  def zero_w_stripe():
    def zfan(j, _):
      pltpu.make_async_copy(
          zrow, w_sp.at[pl.ds(r0 + j * 128, 128)], sem_g).start()
      return 0

    def zdrain(j, _):
      pltpu.make_async_copy(
          zrow, w_sp.at[pl.ds(r0 + j * 128, 128)], sem_g).wait()
      return 0

    lax.fori_loop(0, 48, zfan, 0)
    pltpu.make_async_copy(
        zrow.at[pl.ds(0, 112)], w_sp.at[pl.ds(r0 + 48 * 128, 112)], sem_g
    ).start()
    lax.fori_loop(0, 48, zdrain, 0)
    pltpu.make_async_copy(
        zrow.at[pl.ds(0, 112)], w_sp.at[pl.ds(r0 + 48 * 128, 112)], sem_g
    ).wait()

  def load_dis_chunk(row):
    pltpu.sync_copy(d0.at[pl.ds(row, CH)], t0.at[pl.ds(0, CH)])
    pltpu.sync_copy(d1.at[pl.ds(row, CH)], t1.at[pl.ds(0, CH)])

    def body(i, _):
      v = t0[pl.ds(i * 16, 16)] + t1[pl.ds(i * 16, 16)]
      dbuf[pl.ds(i * 16, 16)] = _rsqrt16(v)
      return 0

    lax.fori_loop(0, CH // 16, body, 0)

  def scale_rows(buf, square):
    def body(i, _):
      for u in range(16):
        rr = i * 16 + u
        d = _splat(dbuf, rr)
        if square:
          d = d * d
        buf[rr] = buf[rr] * d
      return 0

    lax.fori_loop(0, CH // 16, body, 0)

  def init_chunk(j, _):
    row = r0 + j * CH
    load_dis_chunk(row)
    pltpu.sync_copy(x2.at[c, pl.ds(row, CH)], rbuf.at[1, pl.ds(0, CH)])
    scale_rows(rbuf.at[1], False)
    pltpu.sync_copy(rbuf.at[1, pl.ds(0, CH)], zall.at[0, c, pl.ds(row, CH)])
    return 0

  lax.fori_loop(0, NCHV, init_chunk, 0)
  zero_w_stripe()
  plsc.subcore_barrier()

  base = s * TPR
  sems = (sem_s0, sem_s1)
  isems = (sem_i0, sem_i1, sem_i2)

  def run_edge_pass(k):
    def idx_fetch(sg, slot):
      row = 2 * base + 2 * SUP * jnp.where(sg < NSG, sg, 0)
      pltpu.make_async_copy(
          sd2.at[pl.ds(row, 2 * SUP)], ibuf.at[slot], isems[slot]).start()

    def do_sg(sg, slot, drain):
      pltpu.make_async_copy(
          sd2.at[pl.ds(0, 2 * SUP)], ibuf.at[slot], isems[slot]).wait()
      prev = (slot + 2) % 3
      if drain:
        for h in range(2):
          for j in range(G):
            pltpu.make_async_copy(
                rbuf.at[h, pl.ds(j * 128, 128)],
                w_sp.at[ibuf.at[prev, 2 * (h * G + j) + 1]], sems[h]).wait()
      for j in range(2 * G):
        pltpu.make_async_copy(
            zall.at[k, c].at[ibuf.at[slot, 2 * j]],
            rbuf.at[j // G, pl.ds((j % G) * 128, 128)], sem_g).start()
      idx_fetch(sg + 2, prev)
      for j in range(2 * G):
        pltpu.make_async_copy(
            zall.at[k, c].at[ibuf.at[slot, 2 * j]],
            rbuf.at[j // G, pl.ds((j % G) * 128, 128)], sem_g).wait()
      for h in range(2):
        for j in range(G):
          pltpu.make_async_copy(
              rbuf.at[h, pl.ds(j * 128, 128)],
              w_sp.at[ibuf.at[slot, 2 * (h * G + j) + 1]], sems[h]).start(
                  add=True)

    idx_fetch(0, 0)
    idx_fetch(1, 1)
    do_sg(0, 0, False)

    def loop_body(q, _):
      do_sg(3 * q + 1, 1, True)
      do_sg(3 * q + 2, 2, True)
      do_sg(3 * q + 3, 0, True)
      return 0

    lax.fori_loop(0, (NSG - 1) // 3, loop_body, 0)

    for h in range(2):
      for j in range(G):
        pltpu.make_async_copy(
            rbuf.at[h, pl.ds(j * 128, 128)],
            w_sp.at[ibuf.at[0, 2 * (h * G + j) + 1]], sems[h]).wait()
    pltpu.make_async_copy(sd2.at[pl.ds(0, 2 * SUP)], ibuf.at[1], sem_i1).wait()
    pltpu.make_async_copy(sd2.at[pl.ds(0, 2 * SUP)], ibuf.at[2], sem_i2).wait()

  def rescale_to(k):
    def chunk(j, _):
      row = r0 + j * CH
      pltpu.sync_copy(w_sp.at[pl.ds(row, CH)], rbuf.at[0, pl.ds(0, CH)])
      load_dis_chunk(row)
      scale_rows(rbuf.at[0], True)
      pltpu.sync_copy(
          rbuf.at[0, pl.ds(0, CH)], zall.at[k + 1, c, pl.ds(row, CH)])
      return 0

    lax.fori_loop(0, NCHV, chunk, 0)
    zero_w_stripe()

  def hop(k, _):
    run_edge_pass(k)
    plsc.subcore_barrier()
    rescale_to(k)
    plsc.subcore_barrier()
    return 0

  lax.fori_loop(0, 3, hop, 0)

  def final_chunk(j, _):
    row = r0 + j * CH
    pltpu.sync_copy(zall.at[1, c, pl.ds(row, CH)], rbuf.at[0, pl.ds(0, CH)])

    def add_body(i, _):
      for u in range(16):
        rr = i * 16 + u
        rbuf[0, rr] = rbuf[0, rr] + rbuf[1, rr]
      return 0

    def accum(k, _):
      pltpu.sync_copy(zall.at[k, c, pl.ds(row, CH)], rbuf.at[1, pl.ds(0, CH)])
      lax.fori_loop(0, CH // 16, add_body, 0)
      return 0

    lax.fori_loop(2, 4, accum, 0)

    load_dis_chunk(row)

    def dinv_body(i, _):
      sl = pl.ds(i * 16, 16)
      dbuf[sl] = dbuf[sl] * (t0[sl] + t1[sl])
      return 0

    lax.fori_loop(0, CH // 16, dinv_body, 0)

    scale_rows(rbuf.at[0], False)
    pltpu.sync_copy(x2.at[c, pl.ds(row, CH)], rbuf.at[1, pl.ds(0, CH)])

    def avg_body(i, _):
      for u in range(16):
        rr = i * 16 + u
        rbuf[0, rr] = (rbuf[0, rr] + rbuf[1, rr]) * 0.25
      return 0

    lax.fori_loop(0, CH // 16, avg_body, 0)
    pltpu.sync_copy(rbuf.at[0, pl.ds(0, CH)], avg.at[c, pl.ds(row, CH)])
    return 0

  lax.fori_loop(0, NCHV, final_chunk, 0)


def kernel(user_emb, item_emb, edge_index):
  x0 = jnp.concatenate([user_emb, item_emb], axis=0)
  x0 = jnp.pad(x0, ((0, NP - N), (0, 0)))
  x2 = jnp.stack([x0[:, :16], x0[:, 16:]], axis=0)
  pad = jnp.full((EP - E,), PAD_NODE, dtype=_i32)
  src2 = jnp.concatenate([edge_index[0], pad]).reshape(EROWS, 128)
  dst2 = jnp.concatenate([edge_index[1], pad]).reshape(EROWS, 128)
  sd2 = jnp.stack([src2, dst2], axis=1).reshape(2 * EROWS, 128)

  d0, d1 = _deg_kernel(dst2)
  avg, _ = _mega_kernel(sd2, d0, d1, x2)
  avg_full = jnp.concatenate([avg[0], avg[1]], axis=1)
  return avg_full[:N_USERS, :], avg_full[N_USERS:N, :]


# speedup vs baseline: 22.2782x; 1.0837x over previous
import functools

import jax
import jax.numpy as jnp
from jax import lax
from jax.experimental import pallas as pl
from jax.experimental.pallas import tpu as pltpu
from jax.experimental.pallas import tpu_sc as plsc

N_USERS = 60000
N_ITEMS = 40000
N = N_USERS + N_ITEMS
NP = 100096
PAD_NODE = NP - 1
E = 1600000
EP = 1638400
EROWS = EP // 128
TPR = EROWS // 16
SUP = 8
NSG = TPR // SUP
G = 4
DG = 16
DROWS = EROWS // 2
DTPR = DROWS // 16
NGD = DTPR // DG
STRIPE = NP // 16
NCH = 12

_mesh = plsc.VectorSubcoreMesh(core_axis_name="c", subcore_axis_name="s")

_f32 = jnp.float32
_i32 = jnp.int32


def _zero_vec_ref(ref, n):
  zv = jnp.zeros((16,), _f32)

  def body(i, _):
    ref[pl.ds(i * 16, 16)] = zv
    return 0

  lax.fori_loop(0, n // 16, body, 0)


def _rsqrt16(x):
  xh = x * 0.5
  i = plsc.bitcast(x, _i32)
  i = 0x5F3759DF - (i >> 1)
  y = plsc.bitcast(i, _f32)
  y = y * (1.5 - xh * y * y)
  y = y * (1.5 - xh * y * y)
  y = y * (1.5 - xh * y * y)
  return jnp.where(x > 0.0, y, 0.0)


def _splat(ref, i):
  return plsc.load_gather(ref, [jnp.full((16,), i, _i32)])


@functools.partial(
    pl.kernel,
    out_type=(
        jax.ShapeDtypeStruct((NP,), _f32),
        jax.ShapeDtypeStruct((NP,), _f32),
    ),
    mesh=_mesh,
    compiler_params=pltpu.CompilerParams(use_tc_tiling_on_sc=False),
    scratch_types=[
        pltpu.VMEM_SHARED((NP,), _f32),
        pltpu.VMEM((DG, 128), _i32),
        pltpu.VMEM((128,), _f32),
        pltpu.VMEM((512,), _f32),
        pltpu.SemaphoreType.DMA,
    ],
)
def _deg_kernel(dst2, d0_out, d1_out, deg_sp, didx, ones_v, zbuf, sem):
  c = lax.axis_index("c")
  s = lax.axis_index("s")
  r0 = s * STRIPE

  _zero_vec_ref(zbuf, 512)

  def fill_ones(i, _):
    ones_v[pl.ds(i * 16, 16)] = jnp.ones((16,), _f32)
    return 0

  lax.fori_loop(0, 8, fill_ones, 0)

  for j in range(NCH):
    pltpu.make_async_copy(zbuf, deg_sp.at[pl.ds(r0 + j * 512, 512)], sem).start()
  pltpu.make_async_copy(
      zbuf.at[pl.ds(0, 112)], deg_sp.at[pl.ds(r0 + NCH * 512, 112)], sem
  ).start()
  for j in range(NCH):
    pltpu.make_async_copy(zbuf, deg_sp.at[pl.ds(r0 + j * 512, 512)], sem).wait()
  pltpu.make_async_copy(
      zbuf.at[pl.ds(0, 112)], deg_sp.at[pl.ds(r0 + NCH * 512, 112)], sem
  ).wait()

  plsc.subcore_barrier()

  base = c * DROWS + s * DTPR

  def do_group(g, drain):
    if drain:
      for j in range(DG):
        pltpu.make_async_copy(ones_v, deg_sp.at[didx.at[j]], sem).wait()
    pltpu.sync_copy(dst2.at[pl.ds(base + g * DG, DG)], didx)
    for j in range(DG):
      pltpu.make_async_copy(ones_v, deg_sp.at[didx.at[j]], sem).start(add=True)

  do_group(0, False)

  def loop_body(g, _):
    do_group(g, True)
    return 0

  lax.fori_loop(1, NGD, loop_body, 0)

  for j in range(DG):
    pltpu.make_async_copy(ones_v, deg_sp.at[didx.at[j]], sem).wait()

  plsc.subcore_barrier()

  def writeout(dout):
    for j in range(NCH):
      pltpu.sync_copy(deg_sp.at[pl.ds(r0 + j * 512, 512)], zbuf)
      pltpu.sync_copy(zbuf, dout.at[pl.ds(r0 + j * 512, 512)])
    pltpu.sync_copy(deg_sp.at[pl.ds(r0 + NCH * 512, 112)], zbuf.at[pl.ds(0, 112)])
    pltpu.sync_copy(zbuf.at[pl.ds(0, 112)], dout.at[pl.ds(r0 + NCH * 512, 112)])

  @pl.when(c == 0)
  def _():
    writeout(d0_out)

  @pl.when(c == 1)
  def _():
    writeout(d1_out)


CH = 368
NCHV = STRIPE // CH


@functools.partial(
    pl.kernel,
    out_type=(
        jax.ShapeDtypeStruct((NP, 32), _f32),
        jax.ShapeDtypeStruct((4, 2, NP, 16), _f32),
    ),
    mesh=_mesh,
    compiler_params=pltpu.CompilerParams(
        use_tc_tiling_on_sc=False, needs_layout_passes=False),
    scratch_types=[
        pltpu.VMEM_SHARED((NP, 16), _f32),
        pltpu.VMEM((3, 2 * SUP, 128), _i32),
        pltpu.VMEM((2, 512, 16), _f32),
        pltpu.VMEM((128, 16), _f32),
        pltpu.VMEM((512,), _f32),
        pltpu.VMEM((512,), _f32),
        pltpu.VMEM((512,), _f32),
        pltpu.SemaphoreType.DMA,
        pltpu.SemaphoreType.DMA,
        pltpu.SemaphoreType.DMA,
        pltpu.SemaphoreType.DMA,
        pltpu.SemaphoreType.DMA,
        pltpu.SemaphoreType.DMA,
    ],
)
def _mega_kernel(sd2, d0, d1, x0, avg, zall,
                 w_sp, ibuf, rbuf, zrow, t0, t1, dbuf,
                 sem_i0, sem_i1, sem_i2, sem_g, sem_s0, sem_s1):
  c = lax.axis_index("c")
  s = lax.axis_index("s")
  r0 = s * STRIPE

  zv = jnp.zeros((16,), _f32)

  def zbody(i, _):
    zrow[i] = zv
    return 0

  lax.fori_loop(0, 128, zbody, 0)

  def zero_w_stripe():
    def zfan(j, _):
      pltpu.make_async_copy(
          zrow, w_sp.at[pl.ds(r0 + j * 128, 128)], sem_g).start()
      return 0

    def zdrain(j, _):
      pltpu.make_async_copy(
          zrow, w_sp.at[pl.ds(r0 + j * 128, 128)], sem_g).wait()
      return 0

    lax.fori_loop(0, 48, zfan, 0)
    pltpu.make_async_copy(
        zrow.at[pl.ds(0, 112)], w_sp.at[pl.ds(r0 + 48 * 128, 112)], sem_g
    ).start()
    lax.fori_loop(0, 48, zdrain, 0)
    pltpu.make_async_copy(
        zrow.at[pl.ds(0, 112)], w_sp.at[pl.ds(r0 + 48 * 128, 112)], sem_g
    ).wait()

  def load_dis_chunk(row):
    pltpu.sync_copy(d0.at[pl.ds(row, CH)], t0.at[pl.ds(0, CH)])
    pltpu.sync_copy(d1.at[pl.ds(row, CH)], t1.at[pl.ds(0, CH)])

    def body(i, _):
      v = t0[pl.ds(i * 16, 16)] + t1[pl.ds(i * 16, 16)]
      dbuf[pl.ds(i * 16, 16)] = _rsqrt16(v)
      return 0

    lax.fori_loop(0, CH // 16, body, 0)

  def scale_rows(buf, square):
    def body(i, _):
      for u in range(16):
        rr = i * 16 + u
        d = _splat(dbuf, rr)
        if square:
          d = d * d
        buf[rr] = buf[rr] * d
      return 0

    lax.fori_loop(0, CH // 16, body, 0)

  def init_chunk(j, _):
    row = r0 + j * CH
    load_dis_chunk(row)
    pltpu.sync_copy(
        x0.at[pl.ds(row, CH), pl.ds(16 * c, 16)], rbuf.at[1, pl.ds(0, CH)])
    scale_rows(rbuf.at[1], False)
    pltpu.sync_copy(rbuf.at[1, pl.ds(0, CH)], zall.at[0, c, pl.ds(row, CH)])
    return 0

  lax.fori_loop(0, NCHV, init_chunk, 0)
  zero_w_stripe()
  plsc.subcore_barrier()

  base = s * TPR
  sems = (sem_s0, sem_s1)
  isems = (sem_i0, sem_i1, sem_i2)

  def run_edge_pass(k):
    def idx_fetch(sg, slot):
      row = 2 * base + 2 * SUP * jnp.where(sg < NSG, sg, 0)
      pltpu.make_async_copy(
          sd2.at[pl.ds(row, 2 * SUP)], ibuf.at[slot], isems[slot]).start()

    def do_sg(sg, slot, drain):
      pltpu.make_async_copy(
          sd2.at[pl.ds(0, 2 * SUP)], ibuf.at[slot], isems[slot]).wait()
      prev = (slot + 2) % 3
      if drain:
        for h in range(2):
          for j in range(G):
            pltpu.make_async_copy(
                rbuf.at[h, pl.ds(j * 128, 128)],
                w_sp.at[ibuf.at[prev, 2 * (h * G + j) + 1]], sems[h]).wait()
      for j in range(2 * G):
        pltpu.make_async_copy(
            zall.at[k, c].at[ibuf.at[slot, 2 * j]],
            rbuf.at[j // G, pl.ds((j % G) * 128, 128)], sem_g).start()
      idx_fetch(sg + 2, prev)
      for j in range(2 * G):
        pltpu.make_async_copy(
            zall.at[k, c].at[ibuf.at[slot, 2 * j]],
            rbuf.at[j // G, pl.ds((j % G) * 128, 128)], sem_g).wait()
      for h in range(2):
        for j in range(G):
          pltpu.make_async_copy(
              rbuf.at[h, pl.ds(j * 128, 128)],
              w_sp.at[ibuf.at[slot, 2 * (h * G + j) + 1]], sems[h]).start(
                  add=True)

    idx_fetch(0, 0)
    idx_fetch(1, 1)
    do_sg(0, 0, False)

    def loop_body(q, _):
      do_sg(3 * q + 1, 1, True)
      do_sg(3 * q + 2, 2, True)
      do_sg(3 * q + 3, 0, True)
      return 0

    lax.fori_loop(0, (NSG - 1) // 3, loop_body, 0)

    for h in range(2):
      for j in range(G):
        pltpu.make_async_copy(
            rbuf.at[h, pl.ds(j * 128, 128)],
            w_sp.at[ibuf.at[0, 2 * (h * G + j) + 1]], sems[h]).wait()
    pltpu.make_async_copy(sd2.at[pl.ds(0, 2 * SUP)], ibuf.at[1], sem_i1).wait()
    pltpu.make_async_copy(sd2.at[pl.ds(0, 2 * SUP)], ibuf.at[2], sem_i2).wait()

  def rescale_to(k):
    def chunk(j, _):
      row = r0 + j * CH
      pltpu.sync_copy(w_sp.at[pl.ds(row, CH)], rbuf.at[0, pl.ds(0, CH)])
      load_dis_chunk(row)
      scale_rows(rbuf.at[0], True)
      pltpu.sync_copy(
          rbuf.at[0, pl.ds(0, CH)], zall.at[k + 1, c, pl.ds(row, CH)])
      return 0

    lax.fori_loop(0, NCHV, chunk, 0)
    zero_w_stripe()

  def hop(k, _):
    run_edge_pass(k)
    plsc.subcore_barrier()
    rescale_to(k)
    plsc.subcore_barrier()
    return 0

  lax.fori_loop(0, 3, hop, 0)

  def final_chunk(j, _):
    row = r0 + j * CH
    pltpu.sync_copy(zall.at[1, c, pl.ds(row, CH)], rbuf.at[0, pl.ds(0, CH)])

    def add_body(i, _):
      for u in range(16):
        rr = i * 16 + u
        rbuf[0, rr] = rbuf[0, rr] + rbuf[1, rr]
      return 0

    def accum(k, _):
      pltpu.sync_copy(zall.at[k, c, pl.ds(row, CH)], rbuf.at[1, pl.ds(0, CH)])
      lax.fori_loop(0, CH // 16, add_body, 0)
      return 0

    lax.fori_loop(2, 4, accum, 0)

    load_dis_chunk(row)

    def dinv_body(i, _):
      sl = pl.ds(i * 16, 16)
      dbuf[sl] = dbuf[sl] * (t0[sl] + t1[sl])
      return 0

    lax.fori_loop(0, CH // 16, dinv_body, 0)

    scale_rows(rbuf.at[0], False)
    pltpu.sync_copy(
        x0.at[pl.ds(row, CH), pl.ds(16 * c, 16)], rbuf.at[1, pl.ds(0, CH)])

    def avg_body(i, _):
      for u in range(16):
        rr = i * 16 + u
        rbuf[0, rr] = (rbuf[0, rr] + rbuf[1, rr]) * 0.25
      return 0

    lax.fori_loop(0, CH // 16, avg_body, 0)
    pltpu.sync_copy(
        rbuf.at[0, pl.ds(0, CH)], avg.at[pl.ds(row, CH), pl.ds(16 * c, 16)])
    return 0

  lax.fori_loop(0, NCHV, final_chunk, 0)


def kernel(user_emb, item_emb, edge_index):
  x0 = jnp.concatenate([user_emb, item_emb], axis=0)
  x0 = jnp.pad(x0, ((0, NP - N), (0, 0)))
  pad = jnp.full((EP - E,), PAD_NODE, dtype=_i32)
  src2 = jnp.concatenate([edge_index[0], pad]).reshape(EROWS, 128)
  dst2 = jnp.concatenate([edge_index[1], pad]).reshape(EROWS, 128)
  sd2 = jnp.stack([src2, dst2], axis=1).reshape(2 * EROWS, 128)

  d0, d1 = _deg_kernel(dst2)
  avg, _ = _mega_kernel(sd2, d0, d1, x0)
  return avg[:N_USERS, :], avg[N_USERS:N, :]


# speedup vs baseline: 22.3395x; 1.0027x over previous
import functools

import jax
import jax.numpy as jnp
from jax import lax
from jax.experimental import pallas as pl
from jax.experimental.pallas import tpu as pltpu
from jax.experimental.pallas import tpu_sc as plsc

N_USERS = 60000
N_ITEMS = 40000
N = N_USERS + N_ITEMS
NP = 100096
PAD_NODE = NP - 1
E = 1600000
EP = 1638400
EROWS = EP // 128
TPR = EROWS // 16
SUP = 8
NSG = TPR // SUP
G = 4
DG = 16
DROWS = EROWS // 2
DTPR = DROWS // 16
NGD = DTPR // DG
STRIPE = NP // 16
NCH = 12

_mesh = plsc.VectorSubcoreMesh(core_axis_name="c", subcore_axis_name="s")

_f32 = jnp.float32
_i32 = jnp.int32


def _zero_vec_ref(ref, n):
  zv = jnp.zeros((16,), _f32)

  def body(i, _):
    ref[pl.ds(i * 16, 16)] = zv
    return 0

  lax.fori_loop(0, n // 16, body, 0)


def _rsqrt16(x):
  xh = x * 0.5
  i = plsc.bitcast(x, _i32)
  i = 0x5F3759DF - (i >> 1)
  y = plsc.bitcast(i, _f32)
  y = y * (1.5 - xh * y * y)
  y = y * (1.5 - xh * y * y)
  y = y * (1.5 - xh * y * y)
  return jnp.where(x > 0.0, y, 0.0)


def _splat(ref, i):
  return plsc.load_gather(ref, [jnp.full((16,), i, _i32)])


@functools.partial(
    pl.kernel,
    out_type=(
        jax.ShapeDtypeStruct((NP,), _f32),
        jax.ShapeDtypeStruct((NP,), _f32),
    ),
    mesh=_mesh,
    compiler_params=pltpu.CompilerParams(use_tc_tiling_on_sc=False),
    scratch_types=[
        pltpu.VMEM_SHARED((NP,), _f32),
        pltpu.VMEM((DG, 128), _i32),
        pltpu.VMEM((128,), _f32),
        pltpu.VMEM((512,), _f32),
        pltpu.SemaphoreType.DMA,
    ],
)
def _deg_kernel(dst2, d0_out, d1_out, deg_sp, didx, ones_v, zbuf, sem):
  c = lax.axis_index("c")
  s = lax.axis_index("s")
  r0 = s * STRIPE

  _zero_vec_ref(zbuf, 512)

  def fill_ones(i, _):
    ones_v[pl.ds(i * 16, 16)] = jnp.ones((16,), _f32)
    return 0

  lax.fori_loop(0, 8, fill_ones, 0)

  for j in range(NCH):
    pltpu.make_async_copy(zbuf, deg_sp.at[pl.ds(r0 + j * 512, 512)], sem).start()
  pltpu.make_async_copy(
      zbuf.at[pl.ds(0, 112)], deg_sp.at[pl.ds(r0 + NCH * 512, 112)], sem
  ).start()
  for j in range(NCH):
    pltpu.make_async_copy(zbuf, deg_sp.at[pl.ds(r0 + j * 512, 512)], sem).wait()
  pltpu.make_async_copy(
      zbuf.at[pl.ds(0, 112)], deg_sp.at[pl.ds(r0 + NCH * 512, 112)], sem
  ).wait()

  plsc.subcore_barrier()

  base = c * DROWS + s * DTPR

  def do_group(g, drain):
    if drain:
      for j in range(DG):
        pltpu.make_async_copy(ones_v, deg_sp.at[didx.at[j]], sem).wait()
    pltpu.sync_copy(dst2.at[pl.ds(base + g * DG, DG)], didx)
    for j in range(DG):
      pltpu.make_async_copy(ones_v, deg_sp.at[didx.at[j]], sem).start(add=True)

  do_group(0, False)

  def loop_body(g, _):
    do_group(g, True)
    return 0

  lax.fori_loop(1, NGD, loop_body, 0)

  for j in range(DG):
    pltpu.make_async_copy(ones_v, deg_sp.at[didx.at[j]], sem).wait()

  plsc.subcore_barrier()

  def writeout(dout):
    for j in range(NCH):
      pltpu.sync_copy(deg_sp.at[pl.ds(r0 + j * 512, 512)], zbuf)
      pltpu.sync_copy(zbuf, dout.at[pl.ds(r0 + j * 512, 512)])
    pltpu.sync_copy(deg_sp.at[pl.ds(r0 + NCH * 512, 112)], zbuf.at[pl.ds(0, 112)])
    pltpu.sync_copy(zbuf.at[pl.ds(0, 112)], dout.at[pl.ds(r0 + NCH * 512, 112)])

  @pl.when(c == 0)
  def _():
    writeout(d0_out)

  @pl.when(c == 1)
  def _():
    writeout(d1_out)


CH = 368
NCHV = STRIPE // CH


@functools.partial(
    pl.kernel,
    out_type=jax.ShapeDtypeStruct((NP, 32), _f32),
    mesh=_mesh,
    compiler_params=pltpu.CompilerParams(
        use_tc_tiling_on_sc=False, needs_layout_passes=False),
    scratch_types=[
        pltpu.HBM((4, 2, NP, 16), _f32),
        pltpu.VMEM_SHARED((NP, 16), _f32),
        pltpu.VMEM((3, 2 * SUP, 128), _i32),
        pltpu.VMEM((2, 512, 16), _f32),
        pltpu.VMEM((128, 16), _f32),
        pltpu.VMEM((512,), _f32),
        pltpu.VMEM((512,), _f32),
        pltpu.VMEM((512,), _f32),
        pltpu.SemaphoreType.DMA,
        pltpu.SemaphoreType.DMA,
        pltpu.SemaphoreType.DMA,
        pltpu.SemaphoreType.DMA,
        pltpu.SemaphoreType.DMA,
        pltpu.SemaphoreType.DMA,
    ],
)
def _mega_kernel(sd2, d0, d1, x0, avg,
                 zall, w_sp, ibuf, rbuf, zrow, t0, t1, dbuf,
                 sem_i0, sem_i1, sem_i2, sem_g, sem_s0, sem_s1):
  c = lax.axis_index("c")
  s = lax.axis_index("s")
  r0 = s * STRIPE

  zv = jnp.zeros((16,), _f32)

  def zbody(i, _):
    zrow[i] = zv
    return 0

  lax.fori_loop(0, 128, zbody, 0)

  def zero_w_stripe():
    def zfan(j, _):
      pltpu.make_async_copy(
          zrow, w_sp.at[pl.ds(r0 + j * 128, 128)], sem_g).start()
      return 0

    def zdrain(j, _):
      pltpu.make_async_copy(
          zrow, w_sp.at[pl.ds(r0 + j * 128, 128)], sem_g).wait()
      return 0

    lax.fori_loop(0, 48, zfan, 0)
    pltpu.make_async_copy(
        zrow.at[pl.ds(0, 112)], w_sp.at[pl.ds(r0 + 48 * 128, 112)], sem_g
    ).start()
    lax.fori_loop(0, 48, zdrain, 0)
    pltpu.make_async_copy(
        zrow.at[pl.ds(0, 112)], w_sp.at[pl.ds(r0 + 48 * 128, 112)], sem_g
    ).wait()

  def load_dis_chunk(row):
    pltpu.sync_copy(d0.at[pl.ds(row, CH)], t0.at[pl.ds(0, CH)])
    pltpu.sync_copy(d1.at[pl.ds(row, CH)], t1.at[pl.ds(0, CH)])

    def body(i, _):
      v = t0[pl.ds(i * 16, 16)] + t1[pl.ds(i * 16, 16)]
      dbuf[pl.ds(i * 16, 16)] = _rsqrt16(v)
      return 0

    lax.fori_loop(0, CH // 16, body, 0)

  def scale_rows(buf, square):
    def body(i, _):
      for u in range(16):
        rr = i * 16 + u
        d = _splat(dbuf, rr)
        if square:
          d = d * d
        buf[rr] = buf[rr] * d
      return 0

    lax.fori_loop(0, CH // 16, body, 0)

  def init_chunk(j, _):
    row = r0 + j * CH
    load_dis_chunk(row)
    pltpu.sync_copy(
        x0.at[pl.ds(row, CH), pl.ds(16 * c, 16)], rbuf.at[1, pl.ds(0, CH)])
    scale_rows(rbuf.at[1], False)
    pltpu.sync_copy(rbuf.at[1, pl.ds(0, CH)], zall.at[0, c, pl.ds(row, CH)])
    return 0

  lax.fori_loop(0, NCHV, init_chunk, 0)
  zero_w_stripe()
  plsc.subcore_barrier()

  base = s * TPR
  sems = (sem_s0, sem_s1)
  isems = (sem_i0, sem_i1, sem_i2)

  def run_edge_pass(k):
    def idx_fetch(sg, slot):
      row = 2 * base + 2 * SUP * jnp.where(sg < NSG, sg, 0)
      pltpu.make_async_copy(
          sd2.at[pl.ds(row, 2 * SUP)], ibuf.at[slot], isems[slot]).start()

    def do_sg(sg, slot, drain):
      pltpu.make_async_copy(
          sd2.at[pl.ds(0, 2 * SUP)], ibuf.at[slot], isems[slot]).wait()
      prev = (slot + 2) % 3
      if drain:
        for h in range(2):
          for j in range(G):
            pltpu.make_async_copy(
                rbuf.at[h, pl.ds(j * 128, 128)],
                w_sp.at[ibuf.at[prev, 2 * (h * G + j) + 1]], sems[h]).wait()
      for j in range(2 * G):
        pltpu.make_async_copy(
            zall.at[k, c].at[ibuf.at[slot, 2 * j]],
            rbuf.at[j // G, pl.ds((j % G) * 128, 128)], sem_g).start()
      idx_fetch(sg + 2, prev)
      for j in range(2 * G):
        pltpu.make_async_copy(
            zall.at[k, c].at[ibuf.at[slot, 2 * j]],
            rbuf.at[j // G, pl.ds((j % G) * 128, 128)], sem_g).wait()
      for h in range(2):
        for j in range(G):
          pltpu.make_async_copy(
              rbuf.at[h, pl.ds(j * 128, 128)],
              w_sp.at[ibuf.at[slot, 2 * (h * G + j) + 1]], sems[h]).start(
                  add=True)

    idx_fetch(0, 0)
    idx_fetch(1, 1)
    do_sg(0, 0, False)

    def loop_body(q, _):
      do_sg(3 * q + 1, 1, True)
      do_sg(3 * q + 2, 2, True)
      do_sg(3 * q + 3, 0, True)
      return 0

    lax.fori_loop(0, (NSG - 1) // 3, loop_body, 0)

    for h in range(2):
      for j in range(G):
        pltpu.make_async_copy(
            rbuf.at[h, pl.ds(j * 128, 128)],
            w_sp.at[ibuf.at[0, 2 * (h * G + j) + 1]], sems[h]).wait()
    pltpu.make_async_copy(sd2.at[pl.ds(0, 2 * SUP)], ibuf.at[1], sem_i1).wait()
    pltpu.make_async_copy(sd2.at[pl.ds(0, 2 * SUP)], ibuf.at[2], sem_i2).wait()

  def rescale_to(k):
    def chunk(j, _):
      row = r0 + j * CH
      pltpu.sync_copy(w_sp.at[pl.ds(row, CH)], rbuf.at[0, pl.ds(0, CH)])
      load_dis_chunk(row)
      scale_rows(rbuf.at[0], True)
      pltpu.sync_copy(
          rbuf.at[0, pl.ds(0, CH)], zall.at[k + 1, c, pl.ds(row, CH)])
      return 0

    lax.fori_loop(0, NCHV, chunk, 0)
    zero_w_stripe()

  def hop(k, _):
    run_edge_pass(k)
    plsc.subcore_barrier()
    rescale_to(k)
    plsc.subcore_barrier()
    return 0

  lax.fori_loop(0, 3, hop, 0)

  def final_chunk(j, _):
    row = r0 + j * CH
    pltpu.sync_copy(zall.at[1, c, pl.ds(row, CH)], rbuf.at[0, pl.ds(0, CH)])

    def add_body(i, _):
      for u in range(16):
        rr = i * 16 + u
        rbuf[0, rr] = rbuf[0, rr] + rbuf[1, rr]
      return 0

    def accum(k, _):
      pltpu.sync_copy(zall.at[k, c, pl.ds(row, CH)], rbuf.at[1, pl.ds(0, CH)])
      lax.fori_loop(0, CH // 16, add_body, 0)
      return 0

    lax.fori_loop(2, 4, accum, 0)

    load_dis_chunk(row)

    def dinv_body(i, _):
      sl = pl.ds(i * 16, 16)
      dbuf[sl] = dbuf[sl] * (t0[sl] + t1[sl])
      return 0

    lax.fori_loop(0, CH // 16, dinv_body, 0)

    scale_rows(rbuf.at[0], False)
    pltpu.sync_copy(
        x0.at[pl.ds(row, CH), pl.ds(16 * c, 16)], rbuf.at[1, pl.ds(0, CH)])

    def avg_body(i, _):
      for u in range(16):
        rr = i * 16 + u
        rbuf[0, rr] = (rbuf[0, rr] + rbuf[1, rr]) * 0.25
      return 0

    lax.fori_loop(0, CH // 16, avg_body, 0)
    pltpu.sync_copy(
        rbuf.at[0, pl.ds(0, CH)], avg.at[pl.ds(row, CH), pl.ds(16 * c, 16)])
    return 0

  lax.fori_loop(0, NCHV, final_chunk, 0)


def kernel(user_emb, item_emb, edge_index):
  x0 = jnp.concatenate([user_emb, item_emb], axis=0)
  x0 = jnp.pad(x0, ((0, NP - N), (0, 0)))
  pad = jnp.full((EP - E,), PAD_NODE, dtype=_i32)
  src2 = jnp.concatenate([edge_index[0], pad]).reshape(EROWS, 128)
  dst2 = jnp.concatenate([edge_index[1], pad]).reshape(EROWS, 128)
  sd2 = jnp.stack([src2, dst2], axis=1).reshape(2 * EROWS, 128)

  d0, d1 = _deg_kernel(dst2)
  avg = _mega_kernel(sd2, d0, d1, x0)
  return avg[:N_USERS, :], avg[N_USERS:N, :]
